# baseline (speedup 1.0000x reference)
from functools import partial

import jax
import jax.numpy as jnp
from jax.experimental import pallas as pl
from jax.experimental.pallas import tpu as pltpu

VMEM_LIMIT = 32 * 1024 * 1024
MARGIN = 64
ROW_BLOCK = 8
SW = 64


def _fused_block_kernel(x_ref, c1w_ref, c1b_ref, c2c_ref, c2b_ref,
                        c3w_ref, c3b_ref, scw_ref, scb_ref,
                        o_ref, h1e_ref, xbf_ref, *, H, W, dts):
    RB = ROW_BLOCK
    nchunks = H // RB
    CH = RB * W

    h1e_ref[...] = jnp.zeros_like(h1e_ref)

    for c in range(nchunks):
        xc = x_ref[0, pl.ds(c * CH, CH), :].astype(jnp.bfloat16)
        xbf_ref[pl.ds(c * CH, CH), :] = xc
        h1 = jnp.dot(xc, c1w_ref[...], preferred_element_type=jnp.float32)
        h1 = jnp.maximum(h1 + c1b_ref[...], 0.0).astype(jnp.bfloat16)
        for r in range(RB):
            h = c * RB + r
            h1e_ref[pl.ds(MARGIN + (h + 1) * SW, W), :] = \
                h1[r * W:(r + 1) * W, :]

    for c in range(nchunks):
        q0 = MARGIN + (c * RB + 1) * SW
        M = RB * SW
        lhs = jnp.concatenate(
            [h1e_ref[pl.ds(q0 + dt, M), :] for dt in dts], axis=1)
        conv = jnp.dot(lhs, c2c_ref[...], preferred_element_type=jnp.float32)
        h2 = jnp.maximum(conv + c2b_ref[...], 0.0).astype(jnp.bfloat16)

        y = jnp.dot(h2, c3w_ref[...], preferred_element_type=jnp.float32)
        y = jnp.maximum(y + c3b_ref[...], 0.0)

        xc = xbf_ref[pl.ds(c * CH, CH), :]
        s = jnp.dot(xc, scw_ref[...], preferred_element_type=jnp.float32)
        s = jnp.maximum(s + scb_ref[...], 0.0)

        for r in range(RB):
            h = c * RB + r
            o_ref[0, pl.ds(h * W, W), :] = (
                y[r * SW:r * SW + W, :] + s[r * W:(r + 1) * W, :])


def kernel(x, c1_w, c1_b, c2_w, c2_b, c3_w, c3_b, sc_w, sc_b):
    N, H, W, Cin = x.shape
    Cmid = c1_w.shape[1]
    Cout = c3_w.shape[1]

    dts = [(di - 1) * SW + (dj - 1) for di in range(3) for dj in range(3)]
    c2cat = jnp.concatenate(
        [c2_w[di, dj * Cmid:(dj + 1) * Cmid, :]
         for di in range(3) for dj in range(3)], axis=0)

    scratch_rows = 2 * MARGIN + (H + 2) * SW

    xf = x.reshape(N, H * W, Cin)
    out = pl.pallas_call(
        partial(_fused_block_kernel, H=H, W=W, dts=dts),
        out_shape=jax.ShapeDtypeStruct((N, H * W, Cout), jnp.float32),
        grid=(N,),
        in_specs=[
            pl.BlockSpec((1, H * W, Cin), lambda n: (n, 0, 0)),
            pl.BlockSpec((Cin, Cmid), lambda n: (0, 0)),
            pl.BlockSpec((1, Cmid), lambda n: (0, 0)),
            pl.BlockSpec((9 * Cmid, Cmid), lambda n: (0, 0)),
            pl.BlockSpec((1, Cmid), lambda n: (0, 0)),
            pl.BlockSpec((Cmid, Cout), lambda n: (0, 0)),
            pl.BlockSpec((1, Cout), lambda n: (0, 0)),
            pl.BlockSpec((Cin, Cout), lambda n: (0, 0)),
            pl.BlockSpec((1, Cout), lambda n: (0, 0)),
        ],
        out_specs=pl.BlockSpec((1, H * W, Cout), lambda n: (n, 0, 0)),
        scratch_shapes=[
            pltpu.VMEM((scratch_rows, Cmid), jnp.bfloat16),
            pltpu.VMEM((H * W, Cin), jnp.bfloat16),
        ],
        compiler_params=pltpu.CompilerParams(
            dimension_semantics=("parallel",),
            vmem_limit_bytes=VMEM_LIMIT),
    )(xf, c1_w, c1_b.reshape(1, Cmid), c2cat, c2_b.reshape(1, Cmid),
      c3_w, c3_b.reshape(1, Cout), sc_w, sc_b.reshape(1, Cout))
    return out.reshape(N, H, W, Cout)


# speedup vs baseline: 1.3133x; 1.3133x over previous
from functools import partial

import jax
import jax.numpy as jnp
from jax.experimental import pallas as pl
from jax.experimental.pallas import tpu as pltpu

VMEM_LIMIT = 32 * 1024 * 1024
MARGIN = 64
ROW_BLOCK = 8
SW = 64


def _fused_block_kernel(x_ref, c1w_ref, c1b_ref, c2c_ref, c2b_ref,
                        c3w_ref, c3b_ref, scw_ref, scb_ref,
                        o_ref, h1e_ref, xbf_ref, *, H, W, dts):
    RB = ROW_BLOCK
    nchunks = H // RB
    CH = RB * W

    h1e_ref[...] = jnp.zeros_like(h1e_ref)

    for c in range(nchunks):
        xc = x_ref[0, pl.ds(c * CH, CH), :].astype(jnp.bfloat16)
        xbf_ref[pl.ds(c * CH, CH), :] = xc
        h1 = jnp.dot(xc, c1w_ref[...], preferred_element_type=jnp.float32)
        h1 = jnp.maximum(h1 + c1b_ref[...], 0.0).astype(jnp.bfloat16)
        for r in range(RB):
            h = c * RB + r
            h1e_ref[pl.ds(MARGIN + (h + 1) * SW, W), :] = \
                h1[r * W:(r + 1) * W, :]

    for c in range(nchunks):
        q0 = MARGIN + (c * RB + 1) * SW
        M = RB * SW
        conv = None
        for i, (dta, dtb) in enumerate(dts):
            lhs = jnp.concatenate(
                [h1e_ref[pl.ds(q0 + dta, M), :],
                 h1e_ref[pl.ds(q0 + dtb, M), :]], axis=1)
            d = jnp.dot(lhs, c2c_ref[i], preferred_element_type=jnp.float32)
            conv = d if conv is None else conv + d
        h2 = jnp.maximum(conv + c2b_ref[...], 0.0).astype(jnp.bfloat16)

        y = jnp.dot(h2, c3w_ref[...], preferred_element_type=jnp.float32)
        y = jnp.maximum(y + c3b_ref[...], 0.0)

        xc = xbf_ref[pl.ds(c * CH, CH), :]
        s = jnp.dot(xc, scw_ref[...], preferred_element_type=jnp.float32)
        s = jnp.maximum(s + scb_ref[...], 0.0)

        for r in range(RB):
            h = c * RB + r
            o_ref[0, pl.ds(h * W, W), :] = (
                y[r * SW:r * SW + W, :] + s[r * W:(r + 1) * W, :])


def kernel(x, c1_w, c1_b, c2_w, c2_b, c3_w, c3_b, sc_w, sc_b):
    N, H, W, Cin = x.shape
    Cmid = c1_w.shape[1]
    Cout = c3_w.shape[1]

    offs = [(di - 1) * SW + (dj - 1) for di in range(3) for dj in range(3)]
    taps = [c2_w[di, dj * Cmid:(dj + 1) * Cmid, :]
            for di in range(3) for dj in range(3)]
    c2cat = jnp.stack([
        jnp.concatenate([taps[0], taps[1]], axis=0),
        jnp.concatenate([taps[2], taps[3]], axis=0),
        jnp.concatenate([taps[4], taps[5]], axis=0),
        jnp.concatenate([taps[6], taps[7]], axis=0),
        jnp.concatenate([taps[8], jnp.zeros_like(taps[8])], axis=0),
    ])
    dts = [(offs[0], offs[1]), (offs[2], offs[3]), (offs[4], offs[5]),
           (offs[6], offs[7]), (offs[8], offs[8])]

    scratch_rows = 2 * MARGIN + (H + 2) * SW

    xf = x.reshape(N, H * W, Cin)
    out = pl.pallas_call(
        partial(_fused_block_kernel, H=H, W=W, dts=dts),
        out_shape=jax.ShapeDtypeStruct((N, H * W, Cout), jnp.float32),
        grid=(N,),
        in_specs=[
            pl.BlockSpec((1, H * W, Cin), lambda n: (n, 0, 0)),
            pl.BlockSpec((Cin, Cmid), lambda n: (0, 0)),
            pl.BlockSpec((1, Cmid), lambda n: (0, 0)),
            pl.BlockSpec((5, 2 * Cmid, Cmid), lambda n: (0, 0, 0)),
            pl.BlockSpec((1, Cmid), lambda n: (0, 0)),
            pl.BlockSpec((Cmid, Cout), lambda n: (0, 0)),
            pl.BlockSpec((1, Cout), lambda n: (0, 0)),
            pl.BlockSpec((Cin, Cout), lambda n: (0, 0)),
            pl.BlockSpec((1, Cout), lambda n: (0, 0)),
        ],
        out_specs=pl.BlockSpec((1, H * W, Cout), lambda n: (n, 0, 0)),
        scratch_shapes=[
            pltpu.VMEM((scratch_rows, Cmid), jnp.bfloat16),
            pltpu.VMEM((H * W, Cin), jnp.bfloat16),
        ],
        compiler_params=pltpu.CompilerParams(
            dimension_semantics=("parallel",),
            vmem_limit_bytes=VMEM_LIMIT),
    )(xf, c1_w, c1_b.reshape(1, Cmid), c2cat, c2_b.reshape(1, Cmid),
      c3_w, c3_b.reshape(1, Cout), sc_w, sc_b.reshape(1, Cout))
    return out.reshape(N, H, W, Cout)


# speedup vs baseline: 1.3498x; 1.0278x over previous
from functools import partial

import jax
import jax.numpy as jnp
from jax.experimental import pallas as pl
from jax.experimental.pallas import tpu as pltpu

VMEM_LIMIT = 32 * 1024 * 1024
MARGIN = 64
ROW_BLOCK = 8
SUB = 2
SW = 64


def _fused_block_kernel(x_ref, c1w_ref, c1b_ref, c2c_ref, c2b_ref,
                        c3w_ref, c3b_ref, scw_ref, scb_ref,
                        o_ref, b0_ref, b1_ref, b2_ref, xbf_ref, *, H, W):
    RB = ROW_BLOCK
    nchunks = H // RB
    CH = RB * W
    bufs = [b0_ref, b1_ref, b2_ref]

    @pl.when(pl.program_id(0) == 0)
    def _():
        b0_ref[...] = jnp.zeros_like(b0_ref)
        b1_ref[...] = jnp.zeros_like(b1_ref)
        b2_ref[...] = jnp.zeros_like(b2_ref)

    for c in range(nchunks):
        xc = x_ref[0, pl.ds(c * CH, CH), :].astype(jnp.bfloat16)
        xbf_ref[pl.ds(c * CH, CH), :] = xc
        h1 = jnp.dot(xc, c1w_ref[...], preferred_element_type=jnp.float32)
        h1 = jnp.maximum(h1 + c1b_ref[...], 0.0).astype(jnp.bfloat16)
        for r in range(RB):
            base = MARGIN + (c * RB + r + 1) * SW
            row = h1[r * W:(r + 1) * W, :]
            b0_ref[pl.ds(base + 1, W), :] = row
            b1_ref[pl.ds(base, W), :] = row
            b2_ref[pl.ds(base - 1, W), :] = row

    nsub = RB // SUB
    M = SUB * SW
    for c in range(nchunks):
        convs = []
        for g in range(nsub):
            q0 = MARGIN + (c * RB + g * SUB + 1) * SW
            lhs = jnp.concatenate(
                [bufs[dj][pl.ds(q0 + (di - 1) * SW, M), :]
                 for di in range(3) for dj in range(3)], axis=1)
            convs.append(jnp.dot(lhs, c2c_ref[...],
                                 preferred_element_type=jnp.float32))
        conv = jnp.concatenate(convs, axis=0)
        h2 = jnp.maximum(conv + c2b_ref[...], 0.0).astype(jnp.bfloat16)

        y = jnp.dot(h2, c3w_ref[...], preferred_element_type=jnp.float32)
        y = jnp.maximum(y + c3b_ref[...], 0.0)

        xc = xbf_ref[pl.ds(c * CH, CH), :]
        s = jnp.dot(xc, scw_ref[...], preferred_element_type=jnp.float32)
        s = jnp.maximum(s + scb_ref[...], 0.0)

        for r in range(RB):
            h = c * RB + r
            o_ref[0, pl.ds(h * W, W), :] = (
                y[r * SW:r * SW + W, :] + s[r * W:(r + 1) * W, :])


def kernel(x, c1_w, c1_b, c2_w, c2_b, c3_w, c3_b, sc_w, sc_b):
    N, H, W, Cin = x.shape
    Cmid = c1_w.shape[1]
    Cout = c3_w.shape[1]

    c2cat = jnp.concatenate(
        [c2_w[di, dj * Cmid:(dj + 1) * Cmid, :]
         for di in range(3) for dj in range(3)], axis=0)

    scratch_rows = 2 * MARGIN + (H + 2) * SW

    xf = x.reshape(N, H * W, Cin)
    out = pl.pallas_call(
        partial(_fused_block_kernel, H=H, W=W),
        out_shape=jax.ShapeDtypeStruct((N, H * W, Cout), jnp.float32),
        grid=(N,),
        in_specs=[
            pl.BlockSpec((1, H * W, Cin), lambda n: (n, 0, 0)),
            pl.BlockSpec((Cin, Cmid), lambda n: (0, 0)),
            pl.BlockSpec((1, Cmid), lambda n: (0, 0)),
            pl.BlockSpec((9 * Cmid, Cmid), lambda n: (0, 0)),
            pl.BlockSpec((1, Cmid), lambda n: (0, 0)),
            pl.BlockSpec((Cmid, Cout), lambda n: (0, 0)),
            pl.BlockSpec((1, Cout), lambda n: (0, 0)),
            pl.BlockSpec((Cin, Cout), lambda n: (0, 0)),
            pl.BlockSpec((1, Cout), lambda n: (0, 0)),
        ],
        out_specs=pl.BlockSpec((1, H * W, Cout), lambda n: (n, 0, 0)),
        scratch_shapes=[
            pltpu.VMEM((scratch_rows, Cmid), jnp.bfloat16),
            pltpu.VMEM((scratch_rows, Cmid), jnp.bfloat16),
            pltpu.VMEM((scratch_rows, Cmid), jnp.bfloat16),
            pltpu.VMEM((H * W, Cin), jnp.bfloat16),
        ],
        compiler_params=pltpu.CompilerParams(
            dimension_semantics=("arbitrary",),
            vmem_limit_bytes=VMEM_LIMIT),
    )(xf, c1_w, c1_b.reshape(1, Cmid), c2cat, c2_b.reshape(1, Cmid),
      c3_w, c3_b.reshape(1, Cout), sc_w, sc_b.reshape(1, Cout))
    return out.reshape(N, H, W, Cout)
